# baseline (speedup 1.0000x reference)
import functools

import jax
import jax.numpy as jnp
import numpy as np
from jax import lax
from jax.experimental import pallas as pl
from jax.experimental.pallas import tpu as pltpu
from jax.experimental.pallas import tpu_sc as plsc

B, N, DIM = 4, 100000, 3
C_DIM = 32
HID = 32
RESO = 32
N_BLOCKS = 5
PADDING = 0.1

BN = B * N
BLK = 4000
NBLK = BN // BLK
BLK_PER_BATCH = N // BLK

_INV = np.float32(1.0 + PADDING + 10e-4)


def _prep_body(p_ref, w_ref, b_ref, w0_ref, b0_ref, w1_ref, b1_ref, ws_ref,
               net_ref, idx_ref):
    p = p_ref[...]
    p_nor = p / _INV + 0.5
    p_nor = jnp.where(p_nor >= 1.0, np.float32(1.0 - 10e-4), p_nor)
    p_nor = jnp.where(p_nor < 0.0, np.float32(0.0), p_nor)
    t = p_nor * RESO
    ti = t.astype(jnp.int32)
    idx = ti[:, 0] + RESO * (ti[:, 1] + RESO * ti[:, 2])
    batch = pl.program_id(0) // BLK_PER_BATCH
    idx_ref[0, 0, :] = idx + batch * (RESO ** 3)
    coord = p_nor - (jnp.floor(t) + 0.5) / RESO
    net = jnp.dot(coord, w_ref[...].T, preferred_element_type=jnp.float32) + b_ref[...]
    h = jnp.dot(jax.nn.relu(net), w0_ref[...].T, preferred_element_type=jnp.float32) + b0_ref[...]
    dx = jnp.dot(jax.nn.relu(h), w1_ref[...].T, preferred_element_type=jnp.float32) + b1_ref[...]
    out = jnp.dot(net, ws_ref[...].T, preferred_element_type=jnp.float32) + dx
    net_ref[...] = jnp.pad(out, ((0, 0), (0, 128 - HID)))


def _full(x):
    nd = x.ndim
    return pl.BlockSpec(x.shape, lambda i: (0,) * nd)


def _prep_call(p_flat, fc_pos_w, fc_pos_b, w0, b0, w1, b1, ws):
    return pl.pallas_call(
        _prep_body,
        grid=(NBLK,),
        in_specs=[pl.BlockSpec((BLK, DIM), lambda i: (i, 0))]
        + [_full(x) for x in (fc_pos_w, fc_pos_b, w0, b0, w1, b1, ws)],
        out_specs=[
            pl.BlockSpec((BLK, 128), lambda i: (i, 0)),
            pl.BlockSpec((1, 1, BLK), lambda i: (i, 0, 0)),
        ],
        out_shape=[
            jax.ShapeDtypeStruct((BN, 128), jnp.float32),
            jax.ShapeDtypeStruct((NBLK, 1, BLK), jnp.int32),
        ],
    )(p_flat, fc_pos_w, fc_pos_b, w0, b0, w1, b1, ws)


def _resnet_body(net_ref, pooled_ref, w0_ref, b0_ref, w1_ref, b1_ref, ws_ref,
                 out_ref):
    x = jnp.concatenate([net_ref[:, :HID], pooled_ref[:, :HID]], axis=1)
    h = jnp.dot(jax.nn.relu(x), w0_ref[...].T, preferred_element_type=jnp.float32) + b0_ref[...]
    dx = jnp.dot(jax.nn.relu(h), w1_ref[...].T, preferred_element_type=jnp.float32) + b1_ref[...]
    out = jnp.dot(x, ws_ref[...].T, preferred_element_type=jnp.float32) + dx
    out_ref[...] = jnp.pad(out, ((0, 0), (0, 128 - HID)))


def _resnet_call(net, pooled, w0, b0, w1, b1, ws):
    return pl.pallas_call(
        _resnet_body,
        grid=(NBLK,),
        in_specs=[
            pl.BlockSpec((BLK, 128), lambda i: (i, 0)),
            pl.BlockSpec((BLK, 128), lambda i: (i, 0)),
        ]
        + [_full(x) for x in (w0, b0, w1, b1, ws)],
        out_specs=pl.BlockSpec((BLK, 128), lambda i: (i, 0)),
        out_shape=jax.ShapeDtypeStruct((BN, 128), jnp.float32),
    )(net, pooled, w0, b0, w1, b1, ws)


def _final_body(net_ref, w_ref, b_ref, out_ref):
    i = pl.program_id(0)
    c = jnp.dot(net_ref[:, :HID], w_ref[...].T, preferred_element_type=jnp.float32) + b_ref[...]
    part = jnp.sum(c, axis=0, keepdims=True)

    @pl.when(i == 0)
    def _():
        out_ref[...] = jnp.zeros_like(out_ref)

    rowmask = jax.lax.broadcasted_iota(jnp.int32, (B, C_DIM), 0) == (i // BLK_PER_BATCH)
    out_ref[...] += jnp.where(rowmask, part, np.float32(0.0))


def _final_call(net, fc_c_w, fc_c_b):
    return pl.pallas_call(
        _final_body,
        grid=(NBLK,),
        in_specs=[pl.BlockSpec((BLK, 128), lambda i: (i, 0)),
                  _full(fc_c_w), _full(fc_c_b)],
        out_specs=pl.BlockSpec((B, C_DIM), lambda i: (0, 0)),
        out_shape=jax.ShapeDtypeStruct((B, C_DIM), jnp.float32),
    )(net, fc_c_w, fc_c_b)


NBKT = 64
BKT_VOX = 2048
CAP = 12800
LCAP = 16896
SCH = 400
PCH = 384
POS_MASK = (1 << 19) - 1

_IOTA = lambda: lax.iota(jnp.int32, 16)


def _dg(x, idx):
    return lax.gather(
        x, idx[:, None],
        dimension_numbers=lax.GatherDimensionNumbers(
            offset_dims=(), collapsed_slice_dims=(0,), start_index_map=(0,)),
        slice_sizes=(1,), mode=lax.GatherScatterMode.PROMISE_IN_BOUNDS)


def _routing_call(gidx):
    mesh = plsc.VectorSubcoreMesh(core_axis_name="c", subcore_axis_name="s")

    @functools.partial(
        pl.kernel,
        mesh=mesh,
        compiler_params=pltpu.CompilerParams(needs_layout_passes=False),
        out_type=[
            jax.ShapeDtypeStruct((2 * 16 * 32 * CAP,), jnp.int32),
            jax.ShapeDtypeStruct((1024,), jnp.int32),
        ],
        scratch_types=[
            pltpu.VMEM((SCH,), jnp.int32),
            pltpu.VMEM((32,), jnp.int32),
            pltpu.VMEM((32,), jnp.int32),
            pltpu.VMEM((LCAP,), jnp.int32),
        ],
    )
    def k(gidx_hbm, routed_hbm, counts_hbm, gv, hist, cur, lbuf):
        cc = lax.axis_index("c")
        s = lax.axis_index("s")
        iota = _IOTA()
        ones = jnp.ones((16,), jnp.int32)
        k8 = s % 8
        size = jnp.where(k8 < 6, 12800, 11600)
        off = jnp.minimum(k8, 6) * 12800 + jnp.maximum(k8 - 6, 0) * 11600
        base = (2 * cc + s // 8) * N + off
        nch = size // SCH

        hist[pl.ds(0, 16)] = jnp.zeros((16,), jnp.int32)
        hist[pl.ds(16, 16)] = jnp.zeros((16,), jnp.int32)

        def hist_chunk(i, _):
            pltpu.sync_copy(gidx_hbm.at[pl.ds(pl.multiple_of(base + i * SCH, 8), SCH)], gv)
            for j in range(SCH // 16):
                g = gv[pl.ds(j * 16, 16)]
                bk = lax.shift_right_logical(g, 11) - 32 * cc
                bks, _unused = plsc.sort_key_val(bk, bk)
                prev = _dg(bks, jnp.maximum(iota - 1, 0))
                newseg = (bks != prev) | (iota == 0)
                segstart = plsc.cummax(jnp.where(newseg, iota, 0))
                nxt = _dg(bks, jnp.minimum(iota + 1, 15))
                lastm = (bks != nxt) | (iota == 15)
                h = plsc.load_gather(hist, [bks])
                plsc.store_scatter(hist, [bks], h + (iota - segstart) + 1,
                                   mask=lastm)
            return ()

        lax.fori_loop(0, nch, hist_chunk, ())

        h_lo = hist[pl.ds(0, 16)]
        h_hi = hist[pl.ds(16, 16)]
        r_lo = lax.shift_left(lax.shift_right_logical(h_lo + 127, 7), 7)
        r_hi = lax.shift_left(lax.shift_right_logical(h_hi + 127, 7), 7)
        c_lo = plsc.cumsum(r_lo)
        c_hi = plsc.cumsum(r_hi)
        excl_lo = c_lo - r_lo
        tot_lo = jnp.max(c_lo)
        excl_hi = c_hi - r_hi + tot_lo
        cur[pl.ds(0, 16)] = excl_lo
        cur[pl.ds(16, 16)] = excl_hi

        def place_chunk(i, _):
            cbase = base + i * SCH
            pltpu.sync_copy(gidx_hbm.at[pl.ds(pl.multiple_of(cbase, 8), SCH)], gv)
            for j in range(SCH // 16):
                g = gv[pl.ds(j * 16, 16)]
                bk = lax.shift_right_logical(g, 11) - 32 * cc
                lv = g & jnp.int32(BKT_VOX - 1)
                pos = cbase + j * 16 + iota
                rv = lax.shift_left(lv, 19) | pos
                bks, rvs = plsc.sort_key_val(bk, rv)
                prev = _dg(bks, jnp.maximum(iota - 1, 0))
                newseg = (bks != prev) | (iota == 0)
                segstart = plsc.cummax(jnp.where(newseg, iota, 0))
                rank = iota - segstart
                basev = plsc.load_gather(cur, [bks])
                dst = basev + rank
                plsc.store_scatter(lbuf, [dst], rvs)
                nxt = _dg(bks, jnp.minimum(iota + 1, 15))
                lastm = (bks != nxt) | (iota == 15)
                plsc.store_scatter(cur, [bks], dst + 1, mask=lastm)
            return ()

        lax.fori_loop(0, nch, place_chunk, ())

        pltpu.sync_copy(hist, counts_hbm.at[pl.ds(pl.multiple_of((cc * 16 + s) * 32, 8), 32)])

        for bb in range(32):
            half = bb // 16
            lane = bb % 16
            hv = h_lo if half == 0 else h_hi
            ev = excl_lo if half == 0 else excl_hi
            cnt = jnp.max(jnp.where(iota == lane, hv, 0))
            start = jnp.max(jnp.where(iota == lane, ev, 0))
            nblk = lax.shift_right_logical(cnt + 127, 7)

            def wr(i, _, bb=bb, start=start):
                pltpu.sync_copy(
                    lbuf.at[pl.ds(pl.multiple_of(start + i * 128, 8), 128)],
                    routed_hbm.at[pl.ds(pl.multiple_of(
                        ((cc * 16 + s) * 32 + bb) * CAP + i * 128, 8), 128)])
                return ()

            lax.fori_loop(0, nblk, wr, ())

    return k(gidx)


def _pool_round_call(net128, routed, counts):
    mesh = plsc.VectorSubcoreMesh(core_axis_name="c", subcore_axis_name="s")
    NEG = jnp.float32(-jnp.inf)

    @functools.partial(
        pl.kernel,
        mesh=mesh,
        compiler_params=pltpu.CompilerParams(needs_layout_passes=False),
        out_type=jax.ShapeDtypeStruct((BN, 128), jnp.float32),
        scratch_types=[
            pltpu.VMEM((PCH,), jnp.int32),
            pltpu.VMEM((PCH,), jnp.int32),
            pltpu.VMEM((PCH,), jnp.int32),
            pltpu.VMEM((PCH, 128), jnp.float32),
            pltpu.VMEM((BKT_VOX * 16,), jnp.float32),
            pltpu.VMEM((BKT_VOX * 16,), jnp.float32),
            pltpu.VMEM((512,), jnp.int32),
            pltpu.SemaphoreType.DMA,
        ],
    )
    def k(net_hbm, routed_hbm, counts_hbm, out_hbm,
          rvb, posb, lvb, rows, glo, ghi, cvm, sem):
        cc = lax.axis_index("c")
        s = lax.axis_index("s")
        iota = _IOTA()
        iota32 = iota * HID
        pltpu.sync_copy(counts_hbm.at[pl.ds(pl.multiple_of(cc * 512, 8), 512)], cvm)

        def one_pass(pas, _):
            bb_l = s + 16 * pas

            def init_grid(i, _):
                neg = jnp.full((16,), NEG, jnp.float32)
                glo[pl.ds(pl.multiple_of(i * 16, 8), 16)] = neg
                ghi[pl.ds(pl.multiple_of(i * 16, 8), 16)] = neg
                return ()

            lax.fori_loop(0, BKT_VOX * 16 // 16, init_grid, ())

            def load_chunk(sl, kk):
                pltpu.sync_copy(
                    routed_hbm.at[pl.ds(pl.multiple_of(
                        ((cc * 16 + sl) * 32 + bb_l) * CAP + kk * PCH, 8), PCH)], rvb)

            def unpack_chunk(rem):
                def up(j, carry):
                    p0, l0 = carry
                    rv = rvb[pl.ds(pl.multiple_of(j * 16, 8), 16)]
                    pos = jnp.minimum(rv & jnp.int32(POS_MASK),
                                      jnp.int32(BN - 1))
                    lv = lax.shift_right_logical(rv, 19) & jnp.int32(BKT_VOX - 1)
                    p0 = jnp.where(j == 0, _dg(pos, jnp.zeros((16,), jnp.int32)), p0)
                    l0 = jnp.where(j == 0, _dg(lv, jnp.zeros((16,), jnp.int32)), l0)
                    m = (j * 16 + iota) < rem
                    posb[pl.ds(pl.multiple_of(j * 16, 8), 16)] = jnp.where(m, pos, p0)
                    lvb[pl.ds(pl.multiple_of(j * 16, 8), 16)] = jnp.where(m, lv, l0)
                    return (p0, l0)

                z = jnp.zeros((16,), jnp.int32)
                lax.fori_loop(0, PCH // 16, up, (z, z))

            def cnt_of(sl):
                row = cvm[pl.ds(pl.multiple_of(sl * 32 + 16 * pas, 8), 16)]
                return jnp.max(jnp.where(iota == s, row, 0))

            def sub1(sl, _):
                cnt = cnt_of(sl)
                nck = (cnt + PCH - 1) // PCH

                def ph1(kk, _):
                    load_chunk(sl, kk)
                    unpack_chunk(cnt - kk * PCH)
                    pltpu.async_copy(net_hbm.at[posb], rows, sem).wait()

                    def grp(g, _):
                        ids = lvb[pl.ds(pl.multiple_of(g * 16, 8), 16)]
                        for l in range(16):
                            bl = _dg(ids, jnp.full((16,), l, jnp.int32))
                            a = bl * 16 + iota
                            v0 = rows[g * 16 + l, pl.ds(0, 16)]
                            v1 = rows[g * 16 + l, pl.ds(16, 16)]
                            g0 = plsc.load_gather(glo, [a])
                            g1 = plsc.load_gather(ghi, [a])
                            plsc.store_scatter(glo, [a], jnp.maximum(g0, v0))
                            plsc.store_scatter(ghi, [a], jnp.maximum(g1, v1))
                        return ()

                    lax.fori_loop(0, PCH // 16, grp, ())
                    return ()

                lax.fori_loop(0, nck, ph1, ())
                return ()

            lax.fori_loop(0, 16, sub1, ())

            def sub2(sl, _):
                cnt = cnt_of(sl)
                nck = (cnt + PCH - 1) // PCH

                def ph2(kk, _):
                    load_chunk(sl, kk)
                    unpack_chunk(cnt - kk * PCH)

                    def grp(g, _):
                        ids = lvb[pl.ds(pl.multiple_of(g * 16, 8), 16)]
                        for l in range(16):
                            bl = _dg(ids, jnp.full((16,), l, jnp.int32))
                            a = bl * 16 + iota
                            g0 = plsc.load_gather(glo, [a])
                            g1 = plsc.load_gather(ghi, [a])
                            rows[g * 16 + l, pl.ds(0, 16)] = g0
                            rows[g * 16 + l, pl.ds(16, 16)] = g1
                        return ()

                    lax.fori_loop(0, PCH // 16, grp, ())
                    pltpu.async_copy(rows, out_hbm.at[posb], sem).wait()
                    return ()

                lax.fori_loop(0, nck, ph2, ())
                return ()

            lax.fori_loop(0, 16, sub2, ())
            return ()

        lax.fori_loop(0, 2, one_pass, ())

    return k(net128, routed, counts)


def kernel(p, fc_pos_w, fc_pos_b, blk_w0, blk_b0, blk_w1, blk_b1, blk_ws,
           fc_c_w, fc_c_b):
    p_flat = p.reshape(BN, DIM)
    net, idx_blocks = _prep_call(p_flat, fc_pos_w, fc_pos_b,
                                 blk_w0[0], blk_b0[0], blk_w1[0], blk_b1[0],
                                 blk_ws[0])
    gidx = idx_blocks.reshape(BN)
    routed, counts = _routing_call(gidx)
    for i in range(1, N_BLOCKS):
        pooled128 = _pool_round_call(net, routed, counts)
        net = _resnet_call(net, pooled128,
                           blk_w0[i], blk_b0[i], blk_w1[i], blk_b1[i],
                           blk_ws[i])
    c_sum = _final_call(net, fc_c_w, fc_c_b)
    c_mean = c_sum / np.float32(N)
    fea_grid = jnp.zeros((B, C_DIM, RESO, RESO, RESO), jnp.float32)
    fea_grid = fea_grid.at[:, :, 0, 0, 0].set(c_mean)
    mask = jnp.zeros((B, RESO, RESO, RESO), dtype=bool)
    mask = mask.at[:, 0, 0, 0].set(True)
    return fea_grid, mask


# speedup vs baseline: 1.2537x; 1.2537x over previous
import functools

import jax
import jax.numpy as jnp
import numpy as np
from jax import lax
from jax.experimental import pallas as pl
from jax.experimental.pallas import tpu as pltpu
from jax.experimental.pallas import tpu_sc as plsc

B, N, DIM = 4, 100000, 3
C_DIM = 32
HID = 32
RESO = 32
N_BLOCKS = 5
PADDING = 0.1

BN = B * N
BLK = 4000
NBLK = BN // BLK
BLK_PER_BATCH = N // BLK

_INV = np.float32(1.0 + PADDING + 10e-4)


def _prep_body(p_ref, w_ref, b_ref, w0_ref, b0_ref, w1_ref, b1_ref, ws_ref,
               net_ref, idx_ref):
    p = p_ref[...]
    p_nor = p / _INV + 0.5
    p_nor = jnp.where(p_nor >= 1.0, np.float32(1.0 - 10e-4), p_nor)
    p_nor = jnp.where(p_nor < 0.0, np.float32(0.0), p_nor)
    t = p_nor * RESO
    ti = t.astype(jnp.int32)
    idx = ti[:, 0] + RESO * (ti[:, 1] + RESO * ti[:, 2])
    batch = pl.program_id(0) // BLK_PER_BATCH
    idx_ref[0, 0, :] = idx + batch * (RESO ** 3)
    coord = p_nor - (jnp.floor(t) + 0.5) / RESO
    net = jnp.dot(coord, w_ref[...].T, preferred_element_type=jnp.float32) + b_ref[...]
    h = jnp.dot(jax.nn.relu(net), w0_ref[...].T, preferred_element_type=jnp.float32) + b0_ref[...]
    dx = jnp.dot(jax.nn.relu(h), w1_ref[...].T, preferred_element_type=jnp.float32) + b1_ref[...]
    out = jnp.dot(net, ws_ref[...].T, preferred_element_type=jnp.float32) + dx
    net_ref[...] = jnp.pad(out, ((0, 0), (0, 128 - HID)))


def _full(x):
    nd = x.ndim
    return pl.BlockSpec(x.shape, lambda i: (0,) * nd)


def _prep_call(p_flat, fc_pos_w, fc_pos_b, w0, b0, w1, b1, ws):
    return pl.pallas_call(
        _prep_body,
        grid=(NBLK,),
        in_specs=[pl.BlockSpec((BLK, DIM), lambda i: (i, 0))]
        + [_full(x) for x in (fc_pos_w, fc_pos_b, w0, b0, w1, b1, ws)],
        out_specs=[
            pl.BlockSpec((BLK, 128), lambda i: (i, 0)),
            pl.BlockSpec((1, 1, BLK), lambda i: (i, 0, 0)),
        ],
        out_shape=[
            jax.ShapeDtypeStruct((BN, 128), jnp.float32),
            jax.ShapeDtypeStruct((NBLK, 1, BLK), jnp.int32),
        ],
    )(p_flat, fc_pos_w, fc_pos_b, w0, b0, w1, b1, ws)


def _resnet_body(net_ref, pooled_ref, w0_ref, b0_ref, w1_ref, b1_ref, ws_ref,
                 out_ref):
    x = jnp.concatenate([net_ref[:, :HID], pooled_ref[:, :HID]], axis=1)
    h = jnp.dot(jax.nn.relu(x), w0_ref[...].T, preferred_element_type=jnp.float32) + b0_ref[...]
    dx = jnp.dot(jax.nn.relu(h), w1_ref[...].T, preferred_element_type=jnp.float32) + b1_ref[...]
    out = jnp.dot(x, ws_ref[...].T, preferred_element_type=jnp.float32) + dx
    out_ref[...] = jnp.pad(out, ((0, 0), (0, 128 - HID)))


def _resnet_call(net, pooled, w0, b0, w1, b1, ws):
    return pl.pallas_call(
        _resnet_body,
        grid=(NBLK,),
        in_specs=[
            pl.BlockSpec((BLK, 128), lambda i: (i, 0)),
            pl.BlockSpec((BLK, 128), lambda i: (i, 0)),
        ]
        + [_full(x) for x in (w0, b0, w1, b1, ws)],
        out_specs=pl.BlockSpec((BLK, 128), lambda i: (i, 0)),
        out_shape=jax.ShapeDtypeStruct((BN, 128), jnp.float32),
    )(net, pooled, w0, b0, w1, b1, ws)


def _final_body(net_ref, w_ref, b_ref, out_ref):
    i = pl.program_id(0)
    c = jnp.dot(net_ref[:, :HID], w_ref[...].T, preferred_element_type=jnp.float32) + b_ref[...]
    part = jnp.sum(c, axis=0, keepdims=True)

    @pl.when(i == 0)
    def _():
        out_ref[...] = jnp.zeros_like(out_ref)

    rowmask = jax.lax.broadcasted_iota(jnp.int32, (B, C_DIM), 0) == (i // BLK_PER_BATCH)
    out_ref[...] += jnp.where(rowmask, part, np.float32(0.0))


def _final_call(net, fc_c_w, fc_c_b):
    return pl.pallas_call(
        _final_body,
        grid=(NBLK,),
        in_specs=[pl.BlockSpec((BLK, 128), lambda i: (i, 0)),
                  _full(fc_c_w), _full(fc_c_b)],
        out_specs=pl.BlockSpec((B, C_DIM), lambda i: (0, 0)),
        out_shape=jax.ShapeDtypeStruct((B, C_DIM), jnp.float32),
    )(net, fc_c_w, fc_c_b)


NBKT = 64
BKT_VOX = 2048
CAP = 12800
LCAP = 16896
SCH = 400
PCH = 256
POS_MASK = (1 << 19) - 1

_IOTA = lambda: lax.iota(jnp.int32, 16)


def _dg(x, idx):
    return lax.gather(
        x, idx[:, None],
        dimension_numbers=lax.GatherDimensionNumbers(
            offset_dims=(), collapsed_slice_dims=(0,), start_index_map=(0,)),
        slice_sizes=(1,), mode=lax.GatherScatterMode.PROMISE_IN_BOUNDS)


def _routing_call(gidx):
    mesh = plsc.VectorSubcoreMesh(core_axis_name="c", subcore_axis_name="s")

    @functools.partial(
        pl.kernel,
        mesh=mesh,
        compiler_params=pltpu.CompilerParams(needs_layout_passes=False),
        out_type=[
            jax.ShapeDtypeStruct((2 * 16 * 32 * CAP,), jnp.int32),
            jax.ShapeDtypeStruct((1024,), jnp.int32),
        ],
        scratch_types=[
            pltpu.VMEM((SCH,), jnp.int32),
            pltpu.VMEM((32,), jnp.int32),
            pltpu.VMEM((32,), jnp.int32),
            pltpu.VMEM((LCAP,), jnp.int32),
        ],
    )
    def k(gidx_hbm, routed_hbm, counts_hbm, gv, hist, cur, lbuf):
        cc = lax.axis_index("c")
        s = lax.axis_index("s")
        iota = _IOTA()
        ones = jnp.ones((16,), jnp.int32)
        k8 = s % 8
        size = jnp.where(k8 < 6, 12800, 11600)
        off = jnp.minimum(k8, 6) * 12800 + jnp.maximum(k8 - 6, 0) * 11600
        base = (2 * cc + s // 8) * N + off
        nch = size // SCH

        hist[pl.ds(0, 16)] = jnp.zeros((16,), jnp.int32)
        hist[pl.ds(16, 16)] = jnp.zeros((16,), jnp.int32)

        def hist_chunk(i, _):
            pltpu.sync_copy(gidx_hbm.at[pl.ds(pl.multiple_of(base + i * SCH, 8), SCH)], gv)
            for j in range(SCH // 16):
                g = gv[pl.ds(j * 16, 16)]
                bk = lax.shift_right_logical(g, 11) - 32 * cc
                bks, _unused = plsc.sort_key_val(bk, bk)
                prev = _dg(bks, jnp.maximum(iota - 1, 0))
                newseg = (bks != prev) | (iota == 0)
                segstart = plsc.cummax(jnp.where(newseg, iota, 0))
                nxt = _dg(bks, jnp.minimum(iota + 1, 15))
                lastm = (bks != nxt) | (iota == 15)
                h = plsc.load_gather(hist, [bks])
                plsc.store_scatter(hist, [bks], h + (iota - segstart) + 1,
                                   mask=lastm)
            return ()

        lax.fori_loop(0, nch, hist_chunk, ())

        h_lo = hist[pl.ds(0, 16)]
        h_hi = hist[pl.ds(16, 16)]
        r_lo = lax.shift_left(lax.shift_right_logical(h_lo + 127, 7), 7)
        r_hi = lax.shift_left(lax.shift_right_logical(h_hi + 127, 7), 7)
        c_lo = plsc.cumsum(r_lo)
        c_hi = plsc.cumsum(r_hi)
        excl_lo = c_lo - r_lo
        tot_lo = jnp.max(c_lo)
        excl_hi = c_hi - r_hi + tot_lo
        cur[pl.ds(0, 16)] = excl_lo
        cur[pl.ds(16, 16)] = excl_hi

        def place_chunk(i, _):
            cbase = base + i * SCH
            pltpu.sync_copy(gidx_hbm.at[pl.ds(pl.multiple_of(cbase, 8), SCH)], gv)
            for j in range(SCH // 16):
                g = gv[pl.ds(j * 16, 16)]
                bk = lax.shift_right_logical(g, 11) - 32 * cc
                lv = g & jnp.int32(BKT_VOX - 1)
                pos = cbase + j * 16 + iota
                rv = lax.shift_left(lv, 19) | pos
                bks, rvs = plsc.sort_key_val(bk, rv)
                prev = _dg(bks, jnp.maximum(iota - 1, 0))
                newseg = (bks != prev) | (iota == 0)
                segstart = plsc.cummax(jnp.where(newseg, iota, 0))
                rank = iota - segstart
                basev = plsc.load_gather(cur, [bks])
                dst = basev + rank
                plsc.store_scatter(lbuf, [dst], rvs)
                nxt = _dg(bks, jnp.minimum(iota + 1, 15))
                lastm = (bks != nxt) | (iota == 15)
                plsc.store_scatter(cur, [bks], dst + 1, mask=lastm)
            return ()

        lax.fori_loop(0, nch, place_chunk, ())

        pltpu.sync_copy(hist, counts_hbm.at[pl.ds(pl.multiple_of((cc * 16 + s) * 32, 8), 32)])

        for bb in range(32):
            half = bb // 16
            lane = bb % 16
            hv = h_lo if half == 0 else h_hi
            ev = excl_lo if half == 0 else excl_hi
            cnt = jnp.max(jnp.where(iota == lane, hv, 0))
            start = jnp.max(jnp.where(iota == lane, ev, 0))
            nblk = lax.shift_right_logical(cnt + 127, 7)

            def wr(i, _, bb=bb, start=start):
                pltpu.sync_copy(
                    lbuf.at[pl.ds(pl.multiple_of(start + i * 128, 8), 128)],
                    routed_hbm.at[pl.ds(pl.multiple_of(
                        ((cc * 16 + s) * 32 + bb) * CAP + i * 128, 8), 128)])
                return ()

            lax.fori_loop(0, nblk, wr, ())

    return k(gidx)


def _pool_round_call(net128, routed, counts):
    mesh = plsc.VectorSubcoreMesh(core_axis_name="c", subcore_axis_name="s")
    NEG = jnp.float32(-jnp.inf)

    @functools.partial(
        pl.kernel,
        mesh=mesh,
        compiler_params=pltpu.CompilerParams(needs_layout_passes=False),
        out_type=jax.ShapeDtypeStruct((BN, 128), jnp.float32),
        scratch_types=[
            pltpu.VMEM((PCH,), jnp.int32),
            pltpu.VMEM((PCH,), jnp.int32),
            pltpu.VMEM((PCH,), jnp.int32),
            pltpu.VMEM((PCH, 128), jnp.float32),
            pltpu.VMEM((BKT_VOX * 16,), jnp.float32),
            pltpu.VMEM((BKT_VOX * 16,), jnp.float32),
            pltpu.VMEM((512,), jnp.int32),
            pltpu.SemaphoreType.DMA,
        ],
    )
    def k(net_hbm, routed_hbm, counts_hbm, out_hbm,
          rvb, posb, lvb, rows, glo, ghi, cvm, sem):
        cc = lax.axis_index("c")
        s = lax.axis_index("s")
        iota = _IOTA()
        iota32 = iota * HID
        pltpu.sync_copy(counts_hbm.at[pl.ds(pl.multiple_of(cc * 512, 8), 512)], cvm)

        def one_pass(pas, _):
            bb_l = s + 16 * pas

            def init_grid(i, _):
                neg = jnp.full((16,), NEG, jnp.float32)
                glo[pl.ds(pl.multiple_of(i * 16, 8), 16)] = neg
                ghi[pl.ds(pl.multiple_of(i * 16, 8), 16)] = neg
                return ()

            lax.fori_loop(0, BKT_VOX * 16 // 16, init_grid, ())

            def load_chunk(sl, kk):
                pltpu.sync_copy(
                    routed_hbm.at[pl.ds(pl.multiple_of(
                        ((cc * 16 + sl) * 32 + bb_l) * CAP + kk * PCH, 8), PCH)], rvb)

            def unpack_chunk(rem):
                def up(j, carry):
                    p0, l0 = carry
                    rv = rvb[pl.ds(pl.multiple_of(j * 16, 8), 16)]
                    pos = jnp.minimum(rv & jnp.int32(POS_MASK),
                                      jnp.int32(BN - 1))
                    lv = lax.shift_right_logical(rv, 19) & jnp.int32(BKT_VOX - 1)
                    p0 = jnp.where(j == 0, _dg(pos, jnp.zeros((16,), jnp.int32)), p0)
                    l0 = jnp.where(j == 0, _dg(lv, jnp.zeros((16,), jnp.int32)), l0)
                    m = (j * 16 + iota) < rem
                    posb[pl.ds(pl.multiple_of(j * 16, 8), 16)] = jnp.where(m, pos, p0)
                    lvb[pl.ds(pl.multiple_of(j * 16, 8), 16)] = jnp.where(m, lv, l0)
                    return (p0, l0)

                z = jnp.zeros((16,), jnp.int32)
                lax.fori_loop(0, PCH // 16, up, (z, z))

            def cnt_of(sl):
                row = cvm[pl.ds(pl.multiple_of(sl * 32 + 16 * pas, 8), 16)]
                return jnp.max(jnp.where(iota == s, row, 0))

            def sub1(sl, _):
                cnt = cnt_of(sl)
                nck = (cnt + PCH - 1) // PCH

                def ph1(kk, _):
                    load_chunk(sl, kk)
                    unpack_chunk(cnt - kk * PCH)
                    pltpu.async_copy(net_hbm.at[posb], rows, sem).wait()

                    def grp(g, _):
                        ids = lvb[pl.ds(pl.multiple_of(g * 16, 8), 16)]
                        for l in range(16):
                            bl = _dg(ids, jnp.full((16,), l, jnp.int32))
                            a = bl * 16 + iota
                            v0 = rows[g * 16 + l, pl.ds(0, 16)]
                            v1 = rows[g * 16 + l, pl.ds(16, 16)]
                            g0 = plsc.load_gather(glo, [a])
                            g1 = plsc.load_gather(ghi, [a])
                            plsc.store_scatter(glo, [a], jnp.maximum(g0, v0))
                            plsc.store_scatter(ghi, [a], jnp.maximum(g1, v1))
                        return ()

                    lax.fori_loop(0, PCH // 16, grp, ())
                    return ()

                lax.fori_loop(0, nck, ph1, ())
                return ()

            lax.fori_loop(0, 16, sub1, ())

            def sub2(sl, _):
                cnt = cnt_of(sl)
                nck = (cnt + PCH - 1) // PCH

                def ph2(kk, _):
                    load_chunk(sl, kk)
                    unpack_chunk(cnt - kk * PCH)

                    def grp(g, _):
                        ids = lvb[pl.ds(pl.multiple_of(g * 16, 8), 16)]
                        for l in range(16):
                            bl = _dg(ids, jnp.full((16,), l, jnp.int32))
                            a = bl * 16 + iota
                            g0 = plsc.load_gather(glo, [a])
                            g1 = plsc.load_gather(ghi, [a])
                            rows[g * 16 + l, pl.ds(0, 16)] = g0
                            rows[g * 16 + l, pl.ds(16, 16)] = g1
                        return ()

                    lax.fori_loop(0, PCH // 16, grp, ())
                    pltpu.async_copy(rows, out_hbm.at[posb], sem).wait()
                    return ()

                lax.fori_loop(0, nck, ph2, ())
                return ()

            lax.fori_loop(0, 16, sub2, ())
            return ()

        lax.fori_loop(0, 2, one_pass, ())

    return k(net128, routed, counts)


def kernel(p, fc_pos_w, fc_pos_b, blk_w0, blk_b0, blk_w1, blk_b1, blk_ws,
           fc_c_w, fc_c_b):
    p_flat = p.reshape(BN, DIM)
    net, idx_blocks = _prep_call(p_flat, fc_pos_w, fc_pos_b,
                                 blk_w0[0], blk_b0[0], blk_w1[0], blk_b1[0],
                                 blk_ws[0])
    gidx = idx_blocks.reshape(BN)
    routed, counts = _routing_call(gidx)
    for i in range(1, N_BLOCKS):
        pooled128 = _pool_round_call(net, routed, counts)
        net = _resnet_call(net, pooled128,
                           blk_w0[i], blk_b0[i], blk_w1[i], blk_b1[i],
                           blk_ws[i])
    c_sum = _final_call(net, fc_c_w, fc_c_b)
    c_mean = c_sum / np.float32(N)
    fea_grid = jnp.zeros((B, C_DIM, RESO, RESO, RESO), jnp.float32)
    fea_grid = fea_grid.at[:, :, 0, 0, 0].set(c_mean)
    mask = jnp.zeros((B, RESO, RESO, RESO), dtype=bool)
    mask = mask.at[:, 0, 0, 0].set(True)
    return fea_grid, mask
